# baseline (speedup 1.0000x reference)
import functools

import jax
import jax.numpy as jnp
from jax import lax
from jax.experimental import pallas as pl
from jax.experimental.pallas import tpu as pltpu
from jax.experimental.pallas import tpu_sc as plsc

N = 10000
NP = 10240
D = 128
E = 320000
C = 40

NC = 2
NS = 16
NW = NC * NS
CHUNK = 128
NCHW = 80
HALF = NCHW // 2
EP = NW * NCHW * CHUNK
ACCR = 10112
STRIPE = ACCR // NS
OSTR = NP // NS

BLK = 1024
GRID = NP // BLK

_mesh = plsc.VectorSubcoreMesh(core_axis_name="c", subcore_axis_name="s")


@functools.partial(
    pl.kernel,
    mesh=_mesh,
    out_type=jax.ShapeDtypeStruct((NC, NP, D), jnp.float32),
    scratch_types=[
        pltpu.VMEM((NCHW, CHUNK), jnp.int32),
        pltpu.VMEM((CHUNK, D), jnp.float32),
        pltpu.VMEM_SHARED((ACCR, D), jnp.float32),
    ],
)
def _deg_kernel(dst3d_hbm, ones_hbm, zeros_hbm, out_hbm, dst_v, ones_v, acc_sh):
    c = lax.axis_index("c")
    s = lax.axis_index("s")
    wid = s * NC + c
    pltpu.sync_copy(zeros_hbm.at[pl.ds(s * STRIPE, STRIPE)],
                    acc_sh.at[pl.ds(s * STRIPE, STRIPE)])
    pltpu.sync_copy(dst3d_hbm.at[wid], dst_v)
    pltpu.sync_copy(ones_hbm, ones_v)
    plsc.subcore_barrier()

    def body(j, carry):
        pltpu.sync_copy(ones_v, acc_sh.at[dst_v.at[j]], add=True)
        return carry

    lax.fori_loop(0, NCHW, body, 0)
    plsc.subcore_barrier()
    pltpu.sync_copy(acc_sh.at[pl.ds(s * STRIPE, STRIPE)],
                    out_hbm.at[c, pl.ds(s * STRIPE, STRIPE)])


@functools.partial(
    pl.kernel,
    mesh=_mesh,
    out_type=jax.ShapeDtypeStruct((NC, NP, D), jnp.float32),
    scratch_types=[
        pltpu.VMEM((HALF, CHUNK), jnp.int32),
        pltpu.VMEM((HALF, CHUNK), jnp.int32),
        pltpu.VMEM((CHUNK, D), jnp.float32),
        pltpu.VMEM((CHUNK, D), jnp.float32),
        pltpu.VMEM_SHARED((ACCR, D), jnp.float32),
        pltpu.SemaphoreType.DMA,
        pltpu.SemaphoreType.DMA,
    ],
)
def _agg_kernel(src3d_hbm, dst3d_hbm, hs_hbm, zeros_hbm, out_hbm,
                src_v, dst_v, rows_a, rows_b, acc_sh, sem_a, sem_b):
    c = lax.axis_index("c")
    s = lax.axis_index("s")
    wid = s * NC + c
    pltpu.sync_copy(zeros_hbm.at[pl.ds(s * STRIPE, STRIPE)],
                    acc_sh.at[pl.ds(s * STRIPE, STRIPE)])
    plsc.subcore_barrier()

    for h in range(2):
        pltpu.sync_copy(src3d_hbm.at[wid, pl.ds(h * HALF, HALF)], src_v)
        pltpu.sync_copy(dst3d_hbm.at[wid, pl.ds(h * HALF, HALF)], dst_v)
        pltpu.async_copy(hs_hbm.at[src_v.at[0]], rows_a, sem_a)

        def body(i, carry):
            j = 2 * i
            pltpu.make_async_copy(hs_hbm.at[src_v.at[j]], rows_a, sem_a).wait()
            pltpu.async_copy(hs_hbm.at[src_v.at[j + 1]], rows_b, sem_b)
            pltpu.sync_copy(rows_a, acc_sh.at[dst_v.at[j]], add=True)
            pltpu.make_async_copy(
                hs_hbm.at[src_v.at[j + 1]], rows_b, sem_b).wait()

            @pl.when(j + 2 < HALF)
            def _():
                pltpu.async_copy(hs_hbm.at[src_v.at[j + 2]], rows_a, sem_a)

            pltpu.sync_copy(rows_b, acc_sh.at[dst_v.at[j + 1]], add=True)
            return carry

        lax.fori_loop(0, HALF // 2, body, 0)
    plsc.subcore_barrier()
    pltpu.sync_copy(acc_sh.at[pl.ds(s * STRIPE, STRIPE)],
                    out_hbm.at[c, pl.ds(s * STRIPE, STRIPE)])


def _dinv_of(d0_ref, d1_ref):
    deg = d0_ref[:, 0:1] + d1_ref[:, 0:1] + 1.0
    return lax.rsqrt(deg)


def _pre_body(x_ref, w_ref, d0_ref, d1_ref, hs_ref):
    dinv = _dinv_of(d0_ref, d1_ref)
    h = jnp.dot(x_ref[:, :], w_ref[:, :], preferred_element_type=jnp.float32)
    hs_ref[:, :] = h * dinv


def _epi_body(has_res, a_ref, acc0_ref, acc1_ref, hs_ref, d0_ref, d1_ref,
              b_ref, g_ref, be_ref, wn_ref, a_out_ref, hs_out_ref):
    dinv = _dinv_of(d0_ref, d1_ref)
    t = (acc0_ref[:, :] + acc1_ref[:, :] + hs_ref[:, :]) * dinv + b_ref[:, :]
    mu = jnp.mean(t, axis=-1, keepdims=True)
    tc = t - mu
    var = jnp.mean(tc * tc, axis=-1, keepdims=True)
    y = tc * lax.rsqrt(var + 1e-5) * g_ref[:, :] + be_ref[:, :]
    y = jnp.where(y > 0.0, y, jnp.exp(y) - 1.0)
    if has_res:
        y = y + a_ref[:, :]
    a_out_ref[:, :] = y
    hs_out_ref[:, :] = jnp.dot(
        y, wn_ref[:, :], preferred_element_type=jnp.float32) * dinv


def _fin_body(a_ref, acc0_ref, acc1_ref, hs_ref, d0_ref, d1_ref,
              b_ref, g_ref, be_ref, wc_ref, bc_ref, out_ref):
    dinv = _dinv_of(d0_ref, d1_ref)
    t = (acc0_ref[:, :] + acc1_ref[:, :] + hs_ref[:, :]) * dinv + b_ref[:, :]
    mu = jnp.mean(t, axis=-1, keepdims=True)
    tc = t - mu
    var = jnp.mean(tc * tc, axis=-1, keepdims=True)
    y = tc * lax.rsqrt(var + 1e-5) * g_ref[:, :] + be_ref[:, :]
    y = jnp.where(y > 0.0, y, jnp.exp(y) - 1.0)
    y = y + a_ref[:, :]
    out_ref[:, :] = jnp.dot(
        y, wc_ref[:, :], preferred_element_type=jnp.float32) + bc_ref[:, :]


def _row_spec(width):
    return pl.BlockSpec((BLK, width), lambda i: (i, 0))


def _full_spec(r, w):
    return pl.BlockSpec((r, w), lambda i: (0, 0))


def _pre(x, W, deg0, deg1):
    return pl.pallas_call(
        _pre_body,
        grid=(GRID,),
        in_specs=[_row_spec(D), _full_spec(D, D), _row_spec(D), _row_spec(D)],
        out_specs=_row_spec(D),
        out_shape=jax.ShapeDtypeStruct((NP, D), jnp.float32),
    )(x, W, deg0, deg1)


def _epi(has_res, a, acc0, acc1, hs, deg0, deg1, b, g, be, Wn):
    return pl.pallas_call(
        functools.partial(_epi_body, has_res),
        grid=(GRID,),
        in_specs=[_row_spec(D), _row_spec(D), _row_spec(D), _row_spec(D),
                  _row_spec(D), _row_spec(D),
                  _full_spec(1, D), _full_spec(1, D), _full_spec(1, D),
                  _full_spec(D, D)],
        out_specs=(_row_spec(D), _row_spec(D)),
        out_shape=(jax.ShapeDtypeStruct((NP, D), jnp.float32),
                   jax.ShapeDtypeStruct((NP, D), jnp.float32)),
    )(a, acc0, acc1, hs, deg0, deg1, b, g, be, Wn)


def _fin(a, acc0, acc1, hs, deg0, deg1, b, g, be, Wc, bc):
    return pl.pallas_call(
        _fin_body,
        grid=(GRID,),
        in_specs=[_row_spec(D), _row_spec(D), _row_spec(D), _row_spec(D),
                  _row_spec(D), _row_spec(D),
                  _full_spec(1, D), _full_spec(1, D), _full_spec(1, D),
                  _full_spec(D, C), _full_spec(1, C)],
        out_specs=_row_spec(C),
        out_shape=jax.ShapeDtypeStruct((NP, C), jnp.float32),
    )(a, acc0, acc1, hs, deg0, deg1, b, g, be, Wc, bc)


def kernel(x, edge_index, W0, b0, g0, be0, W1, b1, g1, be1,
           W2, b2, g2, be2, Wc, bc):
    pad_e = EP - E
    pad_dst = N + (jnp.arange(pad_e, dtype=jnp.int32) % (ACCR - N))
    src3d = jnp.concatenate(
        [edge_index[0], jnp.zeros((pad_e,), jnp.int32)]).reshape(NW, NCHW, CHUNK)
    dst3d = jnp.concatenate(
        [edge_index[1], pad_dst]).reshape(NW, NCHW, CHUNK)
    ones128 = jnp.ones((CHUNK, D), jnp.float32)
    zeros128 = jnp.zeros((NP, D), jnp.float32)
    xp = jnp.pad(x, ((0, NP - N), (0, 0)))
    b0r, g0r, be0r = b0.reshape(1, D), g0.reshape(1, D), be0.reshape(1, D)
    b1r, g1r, be1r = b1.reshape(1, D), g1.reshape(1, D), be1.reshape(1, D)
    b2r, g2r, be2r = b2.reshape(1, D), g2.reshape(1, D), be2.reshape(1, D)
    bcr = bc.reshape(1, C)

    degp = _deg_kernel(dst3d, ones128, zeros128)
    deg0, deg1 = degp[0], degp[1]

    hs0 = _pre(xp, W0, deg0, deg1)
    accp = _agg_kernel(src3d, dst3d, hs0, zeros128)
    a1, hs1 = _epi(False, xp, accp[0], accp[1], hs0, deg0, deg1,
                   b0r, g0r, be0r, W1)
    accp = _agg_kernel(src3d, dst3d, hs1, zeros128)
    a2, hs2 = _epi(True, a1, accp[0], accp[1], hs1, deg0, deg1,
                   b1r, g1r, be1r, W2)
    accp = _agg_kernel(src3d, dst3d, hs2, zeros128)
    return _fin(a2, accp[0], accp[1], hs2, deg0, deg1,
                b2r, g2r, be2r, Wc, bcr)[:N]


# speedup vs baseline: 2.4099x; 2.4099x over previous
import functools

import jax
import jax.numpy as jnp
from jax import lax
from jax.experimental import pallas as pl
from jax.experimental.pallas import tpu as pltpu
from jax.experimental.pallas import tpu_sc as plsc

N = 10000
NP = 10240
D = 128
E = 320000
C = 40

NC = 2
NS = 16
NW = NC * NS
CHUNK = 128
NCHW = 80
HALF = NCHW // 2
EP = NW * NCHW * CHUNK
ACCR = 10112
STRIPE = ACCR // NS
OSTR = NP // NS

BLK = 1024
GRID = NP // BLK

_mesh = plsc.VectorSubcoreMesh(core_axis_name="c", subcore_axis_name="s")


@functools.partial(
    pl.kernel,
    mesh=_mesh,
    out_type=jax.ShapeDtypeStruct((NC, NP, D), jnp.float32),
    scratch_types=[
        pltpu.VMEM((NCHW, CHUNK), jnp.int32),
        pltpu.VMEM((CHUNK, D), jnp.float32),
        pltpu.VMEM_SHARED((ACCR, D), jnp.float32),
    ],
)
def _deg_kernel(dst3d_hbm, ones_hbm, zeros_hbm, out_hbm, dst_v, ones_v, acc_sh):
    c = lax.axis_index("c")
    s = lax.axis_index("s")
    wid = s * NC + c
    pltpu.sync_copy(zeros_hbm.at[pl.ds(s * STRIPE, STRIPE)],
                    acc_sh.at[pl.ds(s * STRIPE, STRIPE)])
    pltpu.sync_copy(dst3d_hbm.at[wid], dst_v)
    pltpu.sync_copy(ones_hbm, ones_v)
    plsc.subcore_barrier()

    def body(j, carry):
        pltpu.sync_copy(ones_v, acc_sh.at[dst_v.at[j]], add=True)
        return carry

    lax.fori_loop(0, NCHW, body, 0)
    plsc.subcore_barrier()
    pltpu.sync_copy(acc_sh.at[pl.ds(s * STRIPE, STRIPE)],
                    out_hbm.at[c, pl.ds(s * STRIPE, STRIPE)])


@functools.partial(
    pl.kernel,
    mesh=_mesh,
    out_type=jax.ShapeDtypeStruct((NC, NP, D), jnp.float32),
    scratch_types=[
        pltpu.VMEM((HALF, CHUNK), jnp.int32),
        pltpu.VMEM((HALF, CHUNK), jnp.int32),
        pltpu.VMEM((CHUNK, D), jnp.float32),
        pltpu.VMEM((CHUNK, D), jnp.float32),
        pltpu.VMEM_SHARED((ACCR, D), jnp.float32),
        pltpu.SemaphoreType.DMA,
        pltpu.SemaphoreType.DMA,
    ],
)
def _agg_kernel(src3d_hbm, dst3d_hbm, hs_hbm, zeros_hbm, out_hbm,
                src_v, dst_v, rows_a, rows_b, acc_sh, sem_a, sem_b):
    c = lax.axis_index("c")
    s = lax.axis_index("s")
    wid = s * NC + c
    pltpu.sync_copy(zeros_hbm.at[pl.ds(s * STRIPE, STRIPE)],
                    acc_sh.at[pl.ds(s * STRIPE, STRIPE)])
    plsc.subcore_barrier()

    for h in range(2):
        pltpu.sync_copy(src3d_hbm.at[wid, pl.ds(h * HALF, HALF)], src_v)
        pltpu.sync_copy(dst3d_hbm.at[wid, pl.ds(h * HALF, HALF)], dst_v)
        pltpu.async_copy(hs_hbm.at[src_v.at[0]], rows_a, sem_a)

        def body(i, carry):
            j = 2 * i
            pltpu.make_async_copy(hs_hbm.at[src_v.at[j]], rows_a, sem_a).wait()
            pltpu.async_copy(hs_hbm.at[src_v.at[j + 1]], rows_b, sem_b)
            pltpu.sync_copy(rows_a, acc_sh.at[dst_v.at[j]], add=True)
            pltpu.make_async_copy(
                hs_hbm.at[src_v.at[j + 1]], rows_b, sem_b).wait()

            @pl.when(j + 2 < HALF)
            def _():
                pltpu.async_copy(hs_hbm.at[src_v.at[j + 2]], rows_a, sem_a)

            pltpu.sync_copy(rows_b, acc_sh.at[dst_v.at[j + 1]], add=True)
            return carry

        lax.fori_loop(0, HALF // 2, body, 0)
    plsc.subcore_barrier()
    pltpu.sync_copy(acc_sh.at[pl.ds(s * STRIPE, STRIPE)],
                    out_hbm.at[c, pl.ds(s * STRIPE, STRIPE)])


def _dinv_of(d0_ref, d1_ref):
    deg = d0_ref[:, 0:1] + d1_ref[:, 0:1] + 1.0
    return lax.rsqrt(deg)


def _pre_body(x_ref, w_ref, d0_ref, d1_ref, hs_ref):
    dinv = _dinv_of(d0_ref, d1_ref)
    h = jnp.dot(x_ref[:, :], w_ref[:, :], preferred_element_type=jnp.float32)
    hs_ref[:, :] = h * dinv


def _epi_body(has_res, a_ref, acc0_ref, acc1_ref, hs_ref, d0_ref, d1_ref,
              b_ref, g_ref, be_ref, wn_ref, a_out_ref, hs_out_ref):
    dinv = _dinv_of(d0_ref, d1_ref)
    t = (acc0_ref[:, :] + acc1_ref[:, :] + hs_ref[:, :]) * dinv + b_ref[:, :]
    mu = jnp.mean(t, axis=-1, keepdims=True)
    tc = t - mu
    var = jnp.mean(tc * tc, axis=-1, keepdims=True)
    y = tc * lax.rsqrt(var + 1e-5) * g_ref[:, :] + be_ref[:, :]
    y = jnp.where(y > 0.0, y, jnp.exp(y) - 1.0)
    if has_res:
        y = y + a_ref[:, :]
    a_out_ref[:, :] = y
    hs_out_ref[:, :] = jnp.dot(
        y, wn_ref[:, :], preferred_element_type=jnp.float32) * dinv


def _fin_body(a_ref, acc0_ref, acc1_ref, hs_ref, d0_ref, d1_ref,
              b_ref, g_ref, be_ref, wc_ref, bc_ref, out_ref):
    dinv = _dinv_of(d0_ref, d1_ref)
    t = (acc0_ref[:, :] + acc1_ref[:, :] + hs_ref[:, :]) * dinv + b_ref[:, :]
    mu = jnp.mean(t, axis=-1, keepdims=True)
    tc = t - mu
    var = jnp.mean(tc * tc, axis=-1, keepdims=True)
    y = tc * lax.rsqrt(var + 1e-5) * g_ref[:, :] + be_ref[:, :]
    y = jnp.where(y > 0.0, y, jnp.exp(y) - 1.0)
    y = y + a_ref[:, :]
    out_ref[:, :] = jnp.dot(
        y, wc_ref[:, :], preferred_element_type=jnp.float32) + bc_ref[:, :]


def _row_spec(width):
    return pl.BlockSpec((BLK, width), lambda i: (i, 0))


def _full_spec(r, w):
    return pl.BlockSpec((r, w), lambda i: (0, 0))


def _pre(x, W, deg0, deg1):
    return pl.pallas_call(
        _pre_body,
        grid=(GRID,),
        in_specs=[_row_spec(D), _full_spec(D, D), _row_spec(D), _row_spec(D)],
        out_specs=_row_spec(D),
        out_shape=jax.ShapeDtypeStruct((NP, D), jnp.float32),
    )(x, W, deg0, deg1)


def _epi(has_res, a, acc0, acc1, hs, deg0, deg1, b, g, be, Wn):
    return pl.pallas_call(
        functools.partial(_epi_body, has_res),
        grid=(GRID,),
        in_specs=[_row_spec(D), _row_spec(D), _row_spec(D), _row_spec(D),
                  _row_spec(D), _row_spec(D),
                  _full_spec(1, D), _full_spec(1, D), _full_spec(1, D),
                  _full_spec(D, D)],
        out_specs=(_row_spec(D), _row_spec(D)),
        out_shape=(jax.ShapeDtypeStruct((NP, D), jnp.float32),
                   jax.ShapeDtypeStruct((NP, D), jnp.float32)),
    )(a, acc0, acc1, hs, deg0, deg1, b, g, be, Wn)


def _fin(a, acc0, acc1, hs, deg0, deg1, b, g, be, Wc, bc):
    return pl.pallas_call(
        _fin_body,
        grid=(GRID,),
        in_specs=[_row_spec(D), _row_spec(D), _row_spec(D), _row_spec(D),
                  _row_spec(D), _row_spec(D),
                  _full_spec(1, D), _full_spec(1, D), _full_spec(1, D),
                  _full_spec(D, C), _full_spec(1, C)],
        out_specs=_row_spec(C),
        out_shape=jax.ShapeDtypeStruct((NP, C), jnp.float32),
    )(a, acc0, acc1, hs, deg0, deg1, b, g, be, Wc, bc)


def kernel(x, edge_index, W0, b0, g0, be0, W1, b1, g1, be1,
           W2, b2, g2, be2, Wc, bc):
    pad_e = EP - E
    pad_dst = N + (jnp.arange(pad_e, dtype=jnp.int32) % (ACCR - N))
    pad_src = jnp.arange(pad_e, dtype=jnp.int32) % N
    src3d = jnp.concatenate(
        [edge_index[0], pad_src]).reshape(NW, NCHW, CHUNK)
    dst3d = jnp.concatenate(
        [edge_index[1], pad_dst]).reshape(NW, NCHW, CHUNK)
    ones128 = jnp.ones((CHUNK, D), jnp.float32)
    zeros128 = jnp.zeros((NP, D), jnp.float32)
    xp = jnp.pad(x, ((0, NP - N), (0, 0)))
    b0r, g0r, be0r = b0.reshape(1, D), g0.reshape(1, D), be0.reshape(1, D)
    b1r, g1r, be1r = b1.reshape(1, D), g1.reshape(1, D), be1.reshape(1, D)
    b2r, g2r, be2r = b2.reshape(1, D), g2.reshape(1, D), be2.reshape(1, D)
    bcr = bc.reshape(1, C)

    degp = _deg_kernel(dst3d, ones128, zeros128)
    deg0, deg1 = degp[0], degp[1]

    hs0 = _pre(xp, W0, deg0, deg1)
    accp = _agg_kernel(src3d, dst3d, hs0, zeros128)
    a1, hs1 = _epi(False, xp, accp[0], accp[1], hs0, deg0, deg1,
                   b0r, g0r, be0r, W1)
    accp = _agg_kernel(src3d, dst3d, hs1, zeros128)
    a2, hs2 = _epi(True, a1, accp[0], accp[1], hs1, deg0, deg1,
                   b1r, g1r, be1r, W2)
    accp = _agg_kernel(src3d, dst3d, hs2, zeros128)
    return _fin(a2, accp[0], accp[1], hs2, deg0, deg1,
                b2r, g2r, be2r, Wc, bcr)[:N]


# speedup vs baseline: 2.7899x; 1.1577x over previous
import functools

import jax
import jax.numpy as jnp
from jax import lax
from jax.experimental import pallas as pl
from jax.experimental.pallas import tpu as pltpu
from jax.experimental.pallas import tpu_sc as plsc

N = 10000
NP = 10240
D = 128
E = 320000
C = 40

NC = 2
NS = 16
NW = NC * NS
CHUNK = 64
NCHW = 160
SECT = NCHW // 4
EP = NW * NCHW * CHUNK
ACCR = 10112
STRIPE = ACCR // NS
OSTR = NP // NS

BLK = 1024
GRID = NP // BLK

_mesh = plsc.VectorSubcoreMesh(core_axis_name="c", subcore_axis_name="s")


@functools.partial(
    pl.kernel,
    mesh=_mesh,
    out_type=jax.ShapeDtypeStruct((NC, NP, D), jnp.float32),
    scratch_types=[
        pltpu.VMEM((NCHW, CHUNK), jnp.int32),
        pltpu.VMEM((CHUNK, D), jnp.float32),
        pltpu.VMEM_SHARED((ACCR, D), jnp.float32),
    ],
)
def _deg_kernel(dst3d_hbm, ones_hbm, zeros_hbm, out_hbm, dst_v, ones_v, acc_sh):
    c = lax.axis_index("c")
    s = lax.axis_index("s")
    wid = s * NC + c
    pltpu.sync_copy(zeros_hbm.at[pl.ds(s * STRIPE, STRIPE)],
                    acc_sh.at[pl.ds(s * STRIPE, STRIPE)])
    pltpu.sync_copy(dst3d_hbm.at[wid], dst_v)
    pltpu.sync_copy(ones_hbm, ones_v)
    plsc.subcore_barrier()

    def body(j, carry):
        pltpu.sync_copy(ones_v, acc_sh.at[dst_v.at[j]], add=True)
        return carry

    lax.fori_loop(0, NCHW, body, 0)
    plsc.subcore_barrier()
    pltpu.sync_copy(acc_sh.at[pl.ds(s * STRIPE, STRIPE)],
                    out_hbm.at[c, pl.ds(s * STRIPE, STRIPE)])


@functools.partial(
    pl.kernel,
    mesh=_mesh,
    out_type=jax.ShapeDtypeStruct((NC, NP, D), jnp.float32),
    scratch_types=[
        pltpu.VMEM((SECT, CHUNK), jnp.int32),
        pltpu.VMEM((SECT, CHUNK), jnp.int32),
        pltpu.VMEM((CHUNK, D), jnp.float32),
        pltpu.VMEM((CHUNK, D), jnp.float32),
        pltpu.VMEM((CHUNK, D), jnp.float32),
        pltpu.VMEM((CHUNK, D), jnp.float32),
        pltpu.VMEM_SHARED((ACCR, D), jnp.float32),
        pltpu.SemaphoreType.DMA,
        pltpu.SemaphoreType.DMA,
        pltpu.SemaphoreType.DMA,
        pltpu.SemaphoreType.DMA,
    ],
)
def _agg_kernel(src3d_hbm, dst3d_hbm, hs_hbm, zeros_hbm, out_hbm,
                src_v, dst_v, rows_a, rows_b, rows_c, rows_d, acc_sh,
                sem_a, sem_b, sem_c, sem_d):
    c = lax.axis_index("c")
    s = lax.axis_index("s")
    wid = s * NC + c
    pltpu.sync_copy(zeros_hbm.at[pl.ds(s * STRIPE, STRIPE)],
                    acc_sh.at[pl.ds(s * STRIPE, STRIPE)])
    plsc.subcore_barrier()

    bufs = ((rows_a, sem_a), (rows_b, sem_b), (rows_c, sem_c), (rows_d, sem_d))

    for h in range(4):
        pltpu.sync_copy(src3d_hbm.at[wid, pl.ds(h * SECT, SECT)], src_v)
        pltpu.sync_copy(dst3d_hbm.at[wid, pl.ds(h * SECT, SECT)], dst_v)
        for k in range(3):
            pltpu.async_copy(hs_hbm.at[src_v.at[k]], bufs[k][0], bufs[k][1])

        def body(i, carry):
            j = 4 * i
            for k in range(4):
                rv, sm = bufs[k]
                pltpu.make_async_copy(
                    hs_hbm.at[src_v.at[j + k]], rv, sm).wait()
                nj = j + k + 3
                nb = bufs[(k + 3) % 4]

                @pl.when(nj < SECT)
                def _(nj=nj, nb=nb):
                    pltpu.async_copy(hs_hbm.at[src_v.at[nj]], nb[0], nb[1])

                pltpu.sync_copy(rv, acc_sh.at[dst_v.at[j + k]], add=True)
            return carry

        lax.fori_loop(0, SECT // 4, body, 0)
    plsc.subcore_barrier()
    pltpu.sync_copy(acc_sh.at[pl.ds(s * STRIPE, STRIPE)],
                    out_hbm.at[c, pl.ds(s * STRIPE, STRIPE)])


def _dinv_of(d0_ref, d1_ref):
    deg = d0_ref[:, 0:1] + d1_ref[:, 0:1] + 1.0
    return lax.rsqrt(deg)


def _pre_body(x_ref, w_ref, d0_ref, d1_ref, hs_ref):
    dinv = _dinv_of(d0_ref, d1_ref)
    h = jnp.dot(x_ref[:, :], w_ref[:, :], preferred_element_type=jnp.float32)
    hs_ref[:, :] = h * dinv


def _epi_body(has_res, a_ref, acc0_ref, acc1_ref, hs_ref, d0_ref, d1_ref,
              b_ref, g_ref, be_ref, wn_ref, a_out_ref, hs_out_ref):
    dinv = _dinv_of(d0_ref, d1_ref)
    t = (acc0_ref[:, :] + acc1_ref[:, :] + hs_ref[:, :]) * dinv + b_ref[:, :]
    mu = jnp.mean(t, axis=-1, keepdims=True)
    tc = t - mu
    var = jnp.mean(tc * tc, axis=-1, keepdims=True)
    y = tc * lax.rsqrt(var + 1e-5) * g_ref[:, :] + be_ref[:, :]
    y = jnp.where(y > 0.0, y, jnp.exp(y) - 1.0)
    if has_res:
        y = y + a_ref[:, :]
    a_out_ref[:, :] = y
    hs_out_ref[:, :] = jnp.dot(
        y, wn_ref[:, :], preferred_element_type=jnp.float32) * dinv


def _fin_body(a_ref, acc0_ref, acc1_ref, hs_ref, d0_ref, d1_ref,
              b_ref, g_ref, be_ref, wc_ref, bc_ref, out_ref):
    dinv = _dinv_of(d0_ref, d1_ref)
    t = (acc0_ref[:, :] + acc1_ref[:, :] + hs_ref[:, :]) * dinv + b_ref[:, :]
    mu = jnp.mean(t, axis=-1, keepdims=True)
    tc = t - mu
    var = jnp.mean(tc * tc, axis=-1, keepdims=True)
    y = tc * lax.rsqrt(var + 1e-5) * g_ref[:, :] + be_ref[:, :]
    y = jnp.where(y > 0.0, y, jnp.exp(y) - 1.0)
    y = y + a_ref[:, :]
    out_ref[:, :] = jnp.dot(
        y, wc_ref[:, :], preferred_element_type=jnp.float32) + bc_ref[:, :]


def _row_spec(width):
    return pl.BlockSpec((BLK, width), lambda i: (i, 0))


def _full_spec(r, w):
    return pl.BlockSpec((r, w), lambda i: (0, 0))


def _pre(x, W, deg0, deg1):
    return pl.pallas_call(
        _pre_body,
        grid=(GRID,),
        in_specs=[_row_spec(D), _full_spec(D, D), _row_spec(D), _row_spec(D)],
        out_specs=_row_spec(D),
        out_shape=jax.ShapeDtypeStruct((NP, D), jnp.float32),
    )(x, W, deg0, deg1)


def _epi(has_res, a, acc0, acc1, hs, deg0, deg1, b, g, be, Wn):
    return pl.pallas_call(
        functools.partial(_epi_body, has_res),
        grid=(GRID,),
        in_specs=[_row_spec(D), _row_spec(D), _row_spec(D), _row_spec(D),
                  _row_spec(D), _row_spec(D),
                  _full_spec(1, D), _full_spec(1, D), _full_spec(1, D),
                  _full_spec(D, D)],
        out_specs=(_row_spec(D), _row_spec(D)),
        out_shape=(jax.ShapeDtypeStruct((NP, D), jnp.float32),
                   jax.ShapeDtypeStruct((NP, D), jnp.float32)),
    )(a, acc0, acc1, hs, deg0, deg1, b, g, be, Wn)


def _fin(a, acc0, acc1, hs, deg0, deg1, b, g, be, Wc, bc):
    return pl.pallas_call(
        _fin_body,
        grid=(GRID,),
        in_specs=[_row_spec(D), _row_spec(D), _row_spec(D), _row_spec(D),
                  _row_spec(D), _row_spec(D),
                  _full_spec(1, D), _full_spec(1, D), _full_spec(1, D),
                  _full_spec(D, C), _full_spec(1, C)],
        out_specs=_row_spec(C),
        out_shape=jax.ShapeDtypeStruct((NP, C), jnp.float32),
    )(a, acc0, acc1, hs, deg0, deg1, b, g, be, Wc, bc)


def kernel(x, edge_index, W0, b0, g0, be0, W1, b1, g1, be1,
           W2, b2, g2, be2, Wc, bc):
    pad_e = EP - E
    pad_dst = N + (jnp.arange(pad_e, dtype=jnp.int32) % (ACCR - N))
    pad_src = jnp.arange(pad_e, dtype=jnp.int32) % N
    src3d = jnp.concatenate(
        [edge_index[0], pad_src]).reshape(NW, NCHW, CHUNK)
    dst3d = jnp.concatenate(
        [edge_index[1], pad_dst]).reshape(NW, NCHW, CHUNK)
    ones128 = jnp.ones((CHUNK, D), jnp.float32)
    zeros128 = jnp.zeros((NP, D), jnp.float32)
    xp = jnp.pad(x, ((0, NP - N), (0, 0)))
    b0r, g0r, be0r = b0.reshape(1, D), g0.reshape(1, D), be0.reshape(1, D)
    b1r, g1r, be1r = b1.reshape(1, D), g1.reshape(1, D), be1.reshape(1, D)
    b2r, g2r, be2r = b2.reshape(1, D), g2.reshape(1, D), be2.reshape(1, D)
    bcr = bc.reshape(1, C)

    degp = _deg_kernel(dst3d, ones128, zeros128)
    deg0, deg1 = degp[0], degp[1]

    hs0 = _pre(xp, W0, deg0, deg1)
    accp = _agg_kernel(src3d, dst3d, hs0, zeros128)
    a1, hs1 = _epi(False, xp, accp[0], accp[1], hs0, deg0, deg1,
                   b0r, g0r, be0r, W1)
    accp = _agg_kernel(src3d, dst3d, hs1, zeros128)
    a2, hs2 = _epi(True, a1, accp[0], accp[1], hs1, deg0, deg1,
                   b1r, g1r, be1r, W2)
    accp = _agg_kernel(src3d, dst3d, hs2, zeros128)
    return _fin(a2, accp[0], accp[1], hs2, deg0, deg1,
                b2r, g2r, be2r, Wc, bcr)[:N]


# speedup vs baseline: 2.7904x; 1.0002x over previous
import functools

import jax
import jax.numpy as jnp
from jax import lax
from jax.experimental import pallas as pl
from jax.experimental.pallas import tpu as pltpu
from jax.experimental.pallas import tpu_sc as plsc

N = 10000
NP = 10240
D = 128
E = 320000
C = 40

NC = 2
NS = 16
NW = NC * NS
CHUNK = 80
NCHW = 128
SECT = NCHW // 4
EP = NW * NCHW * CHUNK
ACCR = 10112
STRIPE = ACCR // NS
OSTR = NP // NS

BLK = 1024
GRID = NP // BLK

_mesh = plsc.VectorSubcoreMesh(core_axis_name="c", subcore_axis_name="s")


@functools.partial(
    pl.kernel,
    mesh=_mesh,
    out_type=jax.ShapeDtypeStruct((NC, NP, D), jnp.float32),
    scratch_types=[
        pltpu.VMEM((NCHW, CHUNK), jnp.int32),
        pltpu.VMEM((CHUNK, D), jnp.float32),
        pltpu.VMEM_SHARED((ACCR, D), jnp.float32),
    ],
)
def _deg_kernel(dst3d_hbm, ones_hbm, zeros_hbm, out_hbm, dst_v, ones_v, acc_sh):
    c = lax.axis_index("c")
    s = lax.axis_index("s")
    wid = s * NC + c
    pltpu.sync_copy(zeros_hbm.at[pl.ds(s * STRIPE, STRIPE)],
                    acc_sh.at[pl.ds(s * STRIPE, STRIPE)])
    pltpu.sync_copy(dst3d_hbm.at[wid], dst_v)
    pltpu.sync_copy(ones_hbm, ones_v)
    plsc.subcore_barrier()

    def body(j, carry):
        pltpu.sync_copy(ones_v, acc_sh.at[dst_v.at[j]], add=True)
        return carry

    lax.fori_loop(0, NCHW, body, 0)
    plsc.subcore_barrier()
    pltpu.sync_copy(acc_sh.at[pl.ds(s * STRIPE, STRIPE)],
                    out_hbm.at[c, pl.ds(s * STRIPE, STRIPE)])


@functools.partial(
    pl.kernel,
    mesh=_mesh,
    out_type=jax.ShapeDtypeStruct((NC, NP, D), jnp.float32),
    scratch_types=[
        pltpu.VMEM((SECT, CHUNK), jnp.int32),
        pltpu.VMEM((SECT, CHUNK), jnp.int32),
        pltpu.VMEM((CHUNK, D), jnp.float32),
        pltpu.VMEM((CHUNK, D), jnp.float32),
        pltpu.VMEM((CHUNK, D), jnp.float32),
        pltpu.VMEM((CHUNK, D), jnp.float32),
        pltpu.VMEM_SHARED((ACCR, D), jnp.float32),
        pltpu.SemaphoreType.DMA,
        pltpu.SemaphoreType.DMA,
        pltpu.SemaphoreType.DMA,
        pltpu.SemaphoreType.DMA,
    ],
)
def _agg_kernel(src3d_hbm, dst3d_hbm, hs_hbm, zeros_hbm, out_hbm,
                src_v, dst_v, rows_a, rows_b, rows_c, rows_d, acc_sh,
                sem_a, sem_b, sem_c, sem_d):
    c = lax.axis_index("c")
    s = lax.axis_index("s")
    wid = s * NC + c
    pltpu.sync_copy(zeros_hbm.at[pl.ds(s * STRIPE, STRIPE)],
                    acc_sh.at[pl.ds(s * STRIPE, STRIPE)])
    plsc.subcore_barrier()

    bufs = ((rows_a, sem_a), (rows_b, sem_b), (rows_c, sem_c), (rows_d, sem_d))

    for h in range(4):
        pltpu.sync_copy(src3d_hbm.at[wid, pl.ds(h * SECT, SECT)], src_v)
        pltpu.sync_copy(dst3d_hbm.at[wid, pl.ds(h * SECT, SECT)], dst_v)
        for k in range(3):
            pltpu.async_copy(hs_hbm.at[src_v.at[k]], bufs[k][0], bufs[k][1])

        def body(i, carry):
            j = 4 * i
            for k in range(4):
                rv, sm = bufs[k]
                pltpu.make_async_copy(
                    hs_hbm.at[src_v.at[j + k]], rv, sm).wait()
                nj = j + k + 3
                nb = bufs[(k + 3) % 4]

                @pl.when(nj < SECT)
                def _(nj=nj, nb=nb):
                    pltpu.async_copy(hs_hbm.at[src_v.at[nj]], nb[0], nb[1])

                pltpu.sync_copy(rv, acc_sh.at[dst_v.at[j + k]], add=True)
            return carry

        lax.fori_loop(0, SECT // 4, body, 0)
    plsc.subcore_barrier()
    pltpu.sync_copy(acc_sh.at[pl.ds(s * STRIPE, STRIPE)],
                    out_hbm.at[c, pl.ds(s * STRIPE, STRIPE)])


def _dinv_of(d0_ref, d1_ref):
    deg = d0_ref[:, 0:1] + d1_ref[:, 0:1] + 1.0
    return lax.rsqrt(deg)


def _pre_body(x_ref, w_ref, d0_ref, d1_ref, hs_ref):
    dinv = _dinv_of(d0_ref, d1_ref)
    h = jnp.dot(x_ref[:, :], w_ref[:, :], preferred_element_type=jnp.float32)
    hs_ref[:, :] = h * dinv


def _epi_body(has_res, a_ref, acc0_ref, acc1_ref, hs_ref, d0_ref, d1_ref,
              b_ref, g_ref, be_ref, wn_ref, a_out_ref, hs_out_ref):
    dinv = _dinv_of(d0_ref, d1_ref)
    t = (acc0_ref[:, :] + acc1_ref[:, :] + hs_ref[:, :]) * dinv + b_ref[:, :]
    mu = jnp.mean(t, axis=-1, keepdims=True)
    tc = t - mu
    var = jnp.mean(tc * tc, axis=-1, keepdims=True)
    y = tc * lax.rsqrt(var + 1e-5) * g_ref[:, :] + be_ref[:, :]
    y = jnp.where(y > 0.0, y, jnp.exp(y) - 1.0)
    if has_res:
        y = y + a_ref[:, :]
    a_out_ref[:, :] = y
    hs_out_ref[:, :] = jnp.dot(
        y, wn_ref[:, :], preferred_element_type=jnp.float32) * dinv


def _fin_body(a_ref, acc0_ref, acc1_ref, hs_ref, d0_ref, d1_ref,
              b_ref, g_ref, be_ref, wc_ref, bc_ref, out_ref):
    dinv = _dinv_of(d0_ref, d1_ref)
    t = (acc0_ref[:, :] + acc1_ref[:, :] + hs_ref[:, :]) * dinv + b_ref[:, :]
    mu = jnp.mean(t, axis=-1, keepdims=True)
    tc = t - mu
    var = jnp.mean(tc * tc, axis=-1, keepdims=True)
    y = tc * lax.rsqrt(var + 1e-5) * g_ref[:, :] + be_ref[:, :]
    y = jnp.where(y > 0.0, y, jnp.exp(y) - 1.0)
    y = y + a_ref[:, :]
    out_ref[:, :] = jnp.dot(
        y, wc_ref[:, :], preferred_element_type=jnp.float32) + bc_ref[:, :]


def _row_spec(width):
    return pl.BlockSpec((BLK, width), lambda i: (i, 0))


def _full_spec(r, w):
    return pl.BlockSpec((r, w), lambda i: (0, 0))


def _pre(x, W, deg0, deg1):
    return pl.pallas_call(
        _pre_body,
        grid=(GRID,),
        in_specs=[_row_spec(D), _full_spec(D, D), _row_spec(D), _row_spec(D)],
        out_specs=_row_spec(D),
        out_shape=jax.ShapeDtypeStruct((NP, D), jnp.float32),
    )(x, W, deg0, deg1)


def _epi(has_res, a, acc0, acc1, hs, deg0, deg1, b, g, be, Wn):
    return pl.pallas_call(
        functools.partial(_epi_body, has_res),
        grid=(GRID,),
        in_specs=[_row_spec(D), _row_spec(D), _row_spec(D), _row_spec(D),
                  _row_spec(D), _row_spec(D),
                  _full_spec(1, D), _full_spec(1, D), _full_spec(1, D),
                  _full_spec(D, D)],
        out_specs=(_row_spec(D), _row_spec(D)),
        out_shape=(jax.ShapeDtypeStruct((NP, D), jnp.float32),
                   jax.ShapeDtypeStruct((NP, D), jnp.float32)),
    )(a, acc0, acc1, hs, deg0, deg1, b, g, be, Wn)


def _fin(a, acc0, acc1, hs, deg0, deg1, b, g, be, Wc, bc):
    return pl.pallas_call(
        _fin_body,
        grid=(GRID,),
        in_specs=[_row_spec(D), _row_spec(D), _row_spec(D), _row_spec(D),
                  _row_spec(D), _row_spec(D),
                  _full_spec(1, D), _full_spec(1, D), _full_spec(1, D),
                  _full_spec(D, C), _full_spec(1, C)],
        out_specs=_row_spec(C),
        out_shape=jax.ShapeDtypeStruct((NP, C), jnp.float32),
    )(a, acc0, acc1, hs, deg0, deg1, b, g, be, Wc, bc)


def kernel(x, edge_index, W0, b0, g0, be0, W1, b1, g1, be1,
           W2, b2, g2, be2, Wc, bc):
    pad_e = EP - E
    pad_dst = N + (jnp.arange(pad_e, dtype=jnp.int32) % (ACCR - N))
    pad_src = jnp.arange(pad_e, dtype=jnp.int32) % N
    src3d = jnp.concatenate(
        [edge_index[0], pad_src]).reshape(NW, NCHW, CHUNK)
    dst3d = jnp.concatenate(
        [edge_index[1], pad_dst]).reshape(NW, NCHW, CHUNK)
    ones128 = jnp.ones((CHUNK, D), jnp.float32)
    zeros128 = jnp.zeros((NP, D), jnp.float32)
    xp = jnp.pad(x, ((0, NP - N), (0, 0)))
    b0r, g0r, be0r = b0.reshape(1, D), g0.reshape(1, D), be0.reshape(1, D)
    b1r, g1r, be1r = b1.reshape(1, D), g1.reshape(1, D), be1.reshape(1, D)
    b2r, g2r, be2r = b2.reshape(1, D), g2.reshape(1, D), be2.reshape(1, D)
    bcr = bc.reshape(1, C)

    degp = _deg_kernel(dst3d, ones128, zeros128)
    deg0, deg1 = degp[0], degp[1]

    hs0 = _pre(xp, W0, deg0, deg1)
    accp = _agg_kernel(src3d, dst3d, hs0, zeros128)
    a1, hs1 = _epi(False, xp, accp[0], accp[1], hs0, deg0, deg1,
                   b0r, g0r, be0r, W1)
    accp = _agg_kernel(src3d, dst3d, hs1, zeros128)
    a2, hs2 = _epi(True, a1, accp[0], accp[1], hs1, deg0, deg1,
                   b1r, g1r, be1r, W2)
    accp = _agg_kernel(src3d, dst3d, hs2, zeros128)
    return _fin(a2, accp[0], accp[1], hs2, deg0, deg1,
                b2r, g2r, be2r, Wc, bcr)[:N]
